# baseline (speedup 1.0000x reference)
import functools

import jax
import jax.numpy as jnp
from jax import lax
from jax.experimental import pallas as pl
from jax.experimental.pallas import tpu as pltpu
from jax.experimental.pallas import tpu_sc as plsc

D = 32
B = 16384
NC = 2
NS = 16
L = 16
NW = NC * NS
BPW = B // NW
IDXW = 128
NIDX = BPW // IDXW

_mesh = plsc.VectorSubcoreMesh(core_axis_name="c", subcore_axis_name="s")


@functools.partial(
    pl.kernel,
    mesh=_mesh,
    compiler_params=pltpu.CompilerParams(
        needs_layout_passes=False, use_tc_tiling_on_sc=False),
    out_type=jax.ShapeDtypeStruct((B,), jnp.float32),
    scratch_types=[
        pltpu.VMEM((NIDX, IDXW), jnp.int32),
        pltpu.VMEM((NIDX, IDXW), jnp.int32),
        pltpu.VMEM((BPW,), jnp.float32),
        pltpu.VMEM((BPW,), jnp.float32),
        pltpu.VMEM((BPW, D), jnp.float32),
        pltpu.VMEM((BPW, D), jnp.float32),
        pltpu.VMEM((BPW,), jnp.float32),
        pltpu.VMEM((BPW,), jnp.float32),
        pltpu.VMEM((BPW,), jnp.float32),
        pltpu.VMEM((BPW,), jnp.float32),
        pltpu.VMEM((BPW,), jnp.float32),
        pltpu.VMEM((BPW,), jnp.float32),
        pltpu.VMEM((BPW,), jnp.float32),
        pltpu.SemaphoreType.DMA,
        pltpu.SemaphoreType.DMA,
        pltpu.SemaphoreType.DMA,
    ],
)
def _sc_recommender(uidx_hbm, iidx_hbm, bg_hbm, gl_hbm, uemb_hbm, iemb_hbm,
                    t0_hbm, t1_hbm, t2_hbm, t3_hbm, t4_hbm, t5_hbm,
                    out_hbm,
                    uidx_v, iidx_v, bg_v, gl_v, urows_v, irows_v,
                    a0_v, a1_v, a2_v, a3_v, a4_v, a5_v, out_v,
                    sem_u, sem_i, sem_s):
    wid = lax.axis_index("s") * NC + lax.axis_index("c")
    base = wid * BPW

    pltpu.sync_copy(uidx_hbm.at[wid], uidx_v)
    pltpu.sync_copy(iidx_hbm.at[wid], iidx_v)
    pltpu.sync_copy(bg_hbm.at[pl.ds(base, BPW)], bg_v)
    pltpu.sync_copy(gl_hbm.at[pl.ds(base, BPW)], gl_v)

    copies = []
    scalar_dsts = [a0_v, a1_v, a2_v, a3_v, a4_v, a5_v]
    scalar_srcs = [t0_hbm, t1_hbm, t2_hbm, t3_hbm, t4_hbm, t5_hbm]
    for j in range(NIDX):
        sl = pl.ds(j * IDXW, IDXW)
        copies.append(
            pltpu.async_copy(uemb_hbm.at[uidx_v.at[j]], urows_v.at[sl], sem_u))
        copies.append(
            pltpu.async_copy(iemb_hbm.at[iidx_v.at[j]], irows_v.at[sl], sem_i))
        for tbl, dst in zip(scalar_srcs, scalar_dsts):
            copies.append(
                pltpu.async_copy(tbl.at[uidx_v.at[j]], dst.at[sl], sem_s))
    for c in copies:
        c.wait()

    lane = lax.iota(jnp.int32, L)

    def chunk(i, carry):
        o = i * L
        sl = pl.ds(o, L)
        rows = o + lane
        accs = [jnp.zeros((L,), jnp.float32) for _ in range(4)]
        for d in range(D):
            col = jnp.full((L,), d, jnp.int32)
            cu = plsc.load_gather(urows_v, [rows, col])
            cv = plsc.load_gather(irows_v, [rows, col])
            accs[d % 4] = accs[d % 4] + cu * cv
        dot = (accs[0] + accs[1]) + (accs[2] + accs[3])
        pref = dot * 0.2

        bg = bg_v[sl]
        glv = gl_v[sl]
        post = bg + glv * 4.0
        t = post - 110.0
        health = jnp.exp(t * t * (-1.0 / 3200.0))

        a_sel = jnp.where(
            post >= 250.0, a0_v[sl],
            jnp.where(post > 180.0, a1_v[sl],
                      jnp.where(post >= 70.0, a2_v[sl],
                                jnp.where(post >= 55.0, a3_v[sl], a4_v[sl]))))
        logit = a_sel + a5_v[sl]
        alpha = 1.0 / (1.0 + jnp.exp(-logit))
        out_v[sl] = alpha * health + (1.0 - alpha) * pref
        return carry

    lax.fori_loop(0, BPW // L, chunk, 0)

    pltpu.sync_copy(out_v, out_hbm.at[pl.ds(base, BPW)])


def kernel(user_indices, item_indices, blood_glucose, gl, user_emb, item_emb,
           a_hyper2, a_hyper1, a_normal, a_hypo1, a_hypo2, bias):
    ui = user_indices.astype(jnp.int32).reshape(NW, NIDX, IDXW)
    ii = item_indices.astype(jnp.int32).reshape(NW, NIDX, IDXW)
    return _sc_recommender(
        ui, ii, blood_glucose, gl, user_emb, item_emb,
        a_hyper2.reshape(-1), a_hyper1.reshape(-1), a_normal.reshape(-1),
        a_hypo1.reshape(-1), a_hypo2.reshape(-1), bias.reshape(-1))


# speedup vs baseline: 1.2170x; 1.2170x over previous
import functools

import jax
import jax.numpy as jnp
from jax import lax
from jax.experimental import pallas as pl
from jax.experimental.pallas import tpu as pltpu
from jax.experimental.pallas import tpu_sc as plsc

D = 32
B = 16384
NC = 2
NS = 16
L = 16
NW = NC * NS
BPW = B // NW
IDXW = 128
NQ = BPW // IDXW

_mesh = plsc.VectorSubcoreMesh(core_axis_name="c", subcore_axis_name="s")


@functools.partial(
    pl.kernel,
    mesh=_mesh,
    compiler_params=pltpu.CompilerParams(
        needs_layout_passes=False, use_tc_tiling_on_sc=True),
    out_type=jax.ShapeDtypeStruct((B,), jnp.float32),
    scratch_types=[
        pltpu.VMEM((4, IDXW), jnp.int32),
        pltpu.VMEM((4, IDXW), jnp.int32),
        pltpu.VMEM((4, IDXW), jnp.int32),
        pltpu.VMEM((4, IDXW), jnp.int32),
        pltpu.VMEM((BPW,), jnp.float32),
        pltpu.VMEM((BPW,), jnp.float32),
        pltpu.VMEM((IDXW, 4 * D), jnp.float32),
        pltpu.VMEM((IDXW, 4 * D), jnp.float32),
        pltpu.VMEM((IDXW, 4 * D), jnp.float32),
        pltpu.VMEM((IDXW, 4 * D), jnp.float32),
        pltpu.VMEM((BPW,), jnp.float32),
        pltpu.VMEM((BPW,), jnp.float32),
        pltpu.VMEM((BPW,), jnp.float32),
        pltpu.VMEM((BPW,), jnp.float32),
        pltpu.VMEM((BPW,), jnp.float32),
        pltpu.VMEM((BPW,), jnp.float32),
        pltpu.VMEM((BPW,), jnp.float32),
        pltpu.SemaphoreType.DMA,
        pltpu.SemaphoreType.DMA,
        pltpu.SemaphoreType.DMA,
    ],
)
def _sc_recommender(uidx_hbm, iidx_hbm, bg_hbm, gl_hbm, uemb_hbm, iemb_hbm,
                    t0_hbm, t1_hbm, t2_hbm, t3_hbm, t4_hbm, t5_hbm,
                    out_hbm,
                    uidx_v, iidx_v, ublk_v, iblk_v, bg_v, gl_v,
                    urows0_v, urows1_v, irows0_v, irows1_v,
                    a0_v, a1_v, a2_v, a3_v, a4_v, a5_v, out_v,
                    sem_u, sem_i, sem_s):
    wid = lax.axis_index("s") * NC + lax.axis_index("c")
    base = wid * BPW

    pltpu.sync_copy(uidx_hbm.at[wid], uidx_v)
    pltpu.sync_copy(iidx_hbm.at[wid], iidx_v)
    pltpu.sync_copy(bg_hbm.at[pl.ds(base, BPW)], bg_v)
    pltpu.sync_copy(gl_hbm.at[pl.ds(base, BPW)], gl_v)

    for j in range(4):
        for k in range(IDXW // L):
            sl = pl.ds(k * L, L)
            ublk_v[j, sl] = lax.shift_right_logical(uidx_v[j, sl], 2)
            iblk_v[j, sl] = lax.shift_right_logical(iidx_v[j, sl], 2)

    scalar_dsts = [a0_v, a1_v, a2_v, a3_v, a4_v, a5_v]
    scalar_srcs = [t0_hbm, t1_hbm, t2_hbm, t3_hbm, t4_hbm, t5_hbm]
    scopies = []
    for j in range(4):
        sl = pl.ds(j * IDXW, IDXW)
        for tbl, dst in zip(scalar_srcs, scalar_dsts):
            scopies.append(
                pltpu.async_copy(tbl.at[0].at[uidx_v.at[j]], dst.at[sl], sem_s))

    def fire_quarter(q, udst, idst):
        return [
            pltpu.async_copy(uemb_hbm.at[ublk_v.at[q]], udst, sem_u),
            pltpu.async_copy(iemb_hbm.at[iblk_v.at[q]], idst, sem_i),
        ]

    lane = lax.iota(jnp.int32, L)

    def compute_quarter(q, u_h, i_h):
        def chunk(i, carry):
            sl = pl.ds(q * IDXW + i * L, L)
            rows = i * L + lane
            jsl = pl.ds(i * L, L)
            ucol0 = jnp.bitwise_and(uidx_v[q, jsl], 3) * 32
            icol0 = jnp.bitwise_and(iidx_v[q, jsl], 3) * 32
            accs = [jnp.zeros((L,), jnp.float32) for _ in range(4)]
            for d in range(D):
                cu = plsc.load_gather(u_h, [rows, ucol0 + d])
                cv = plsc.load_gather(i_h, [rows, icol0 + d])
                accs[d % 4] = accs[d % 4] + cu * cv
            dot = (accs[0] + accs[1]) + (accs[2] + accs[3])
            pref = dot * 0.2

            bg = bg_v[sl]
            glv = gl_v[sl]
            post = bg + glv * 4.0
            t = post - 110.0
            health = jnp.exp(t * t * (-1.0 / 3200.0))

            a_sel = jnp.where(
                post >= 250.0, a0_v[sl],
                jnp.where(post > 180.0, a1_v[sl],
                          jnp.where(post >= 70.0, a2_v[sl],
                                    jnp.where(post >= 55.0, a3_v[sl],
                                              a4_v[sl]))))
            logit = a_sel + a5_v[sl]
            alpha = 1.0 / (1.0 + jnp.exp(-logit))
            out_v[sl] = alpha * health + (1.0 - alpha) * pref
            return carry

        lax.fori_loop(0, IDXW // L, chunk, 0)

    ubufs = [urows0_v, urows1_v]
    ibufs = [irows0_v, irows1_v]
    cps = [fire_quarter(0, ubufs[0], ibufs[0]),
           fire_quarter(1, ubufs[1], ibufs[1])]
    for c in scopies:
        c.wait()
    for q in range(NQ):
        b = q % 2
        for c in cps[q]:
            c.wait()
        compute_quarter(q, ubufs[b], ibufs[b])
        if q + 2 < NQ:
            cps.append(fire_quarter(q + 2, ubufs[b], ibufs[b]))

    pltpu.sync_copy(out_v, out_hbm.at[pl.ds(base, BPW)])


def kernel(user_indices, item_indices, blood_glucose, gl, user_emb, item_emb,
           a_hyper2, a_hyper1, a_normal, a_hypo1, a_hypo2, bias):
    ui = user_indices.astype(jnp.int32).reshape(NW, 4, IDXW)
    ii = item_indices.astype(jnp.int32).reshape(NW, 4, IDXW)
    uemb = user_emb.reshape(-1, 4 * D)
    iemb = item_emb.reshape(-1, 4 * D)
    return _sc_recommender(
        ui, ii, blood_glucose, gl, uemb, iemb,
        a_hyper2.T, a_hyper1.T, a_normal.T,
        a_hypo1.T, a_hypo2.T, bias.T)


# speedup vs baseline: 1.8928x; 1.5553x over previous
import functools

import jax
import jax.numpy as jnp
from jax import lax
from jax.experimental import pallas as pl
from jax.experimental.pallas import tpu as pltpu
from jax.experimental.pallas import tpu_sc as plsc

D = 32
B = 16384
NC = 2
NS = 16
L = 16
NW = NC * NS
BPW = B // NW
IDXW = 128
NCH = BPW // L
DEPTH = 4

_mesh = plsc.VectorSubcoreMesh(core_axis_name="c", subcore_axis_name="s")


@functools.partial(
    pl.kernel,
    mesh=_mesh,
    compiler_params=pltpu.CompilerParams(
        needs_layout_passes=False, use_tc_tiling_on_sc=True),
    out_type=jax.ShapeDtypeStruct((B,), jnp.float32),
    scratch_types=[
        pltpu.VMEM((BPW,), jnp.int32),
        pltpu.VMEM((BPW,), jnp.int32),
        pltpu.VMEM((BPW,), jnp.float32),
        pltpu.VMEM((BPW,), jnp.float32),
        pltpu.VMEM((DEPTH, L, D), jnp.float32),
        pltpu.VMEM((DEPTH, L, D), jnp.float32),
        pltpu.VMEM((BPW,), jnp.float32),
        pltpu.VMEM((BPW,), jnp.float32),
        pltpu.VMEM((BPW,), jnp.float32),
        pltpu.VMEM((BPW,), jnp.float32),
        pltpu.VMEM((BPW,), jnp.float32),
        pltpu.VMEM((BPW,), jnp.float32),
        pltpu.VMEM((BPW,), jnp.float32),
        pltpu.SemaphoreType.DMA,
        pltpu.SemaphoreType.DMA,
        pltpu.SemaphoreType.DMA,
    ],
)
def _sc_recommender(uidx_hbm, iidx_hbm, bg_hbm, gl_hbm, uemb_hbm, iemb_hbm,
                    t0_hbm, t1_hbm, t2_hbm, t3_hbm, t4_hbm, t5_hbm,
                    out_hbm,
                    uidx_v, iidx_v, bg_v, gl_v, urows_v, irows_v,
                    a0_v, a1_v, a2_v, a3_v, a4_v, a5_v, out_v,
                    sem_u, sem_i, sem_s):
    wid = lax.axis_index("s") * NC + lax.axis_index("c")
    base = wid * BPW

    pltpu.sync_copy(uidx_hbm.at[pl.ds(base, BPW)], uidx_v)
    pltpu.sync_copy(iidx_hbm.at[pl.ds(base, BPW)], iidx_v)
    pltpu.sync_copy(bg_hbm.at[pl.ds(base, BPW)], bg_v)
    pltpu.sync_copy(gl_hbm.at[pl.ds(base, BPW)], gl_v)

    scalar_dsts = [a0_v, a1_v, a2_v, a3_v, a4_v, a5_v]
    scalar_srcs = [t0_hbm, t1_hbm, t2_hbm, t3_hbm, t4_hbm, t5_hbm]
    scopies = []
    for j in range(4):
        sl = pl.ds(j * IDXW, IDXW)
        for tbl, dst in zip(scalar_srcs, scalar_dsts):
            scopies.append(
                pltpu.async_copy(
                    tbl.at[0].at[uidx_v.at[pl.ds(j * IDXW, IDXW)]],
                    dst.at[sl], sem_s))

    def row_copies(c):
        b = lax.bitwise_and(c, DEPTH - 1)
        uvec = uidx_v[pl.ds(c * L, L)]
        ivec = iidx_v[pl.ds(c * L, L)]
        cps = []
        for e in range(L):
            cps.append(pltpu.make_async_copy(
                uemb_hbm.at[uvec[e]], urows_v.at[b].at[e], sem_u))
            cps.append(pltpu.make_async_copy(
                iemb_hbm.at[ivec[e]], irows_v.at[b].at[e], sem_i))
        return cps

    def fire_chunk(c):
        for cp in row_copies(c):
            cp.start()

    def wait_chunk(c):
        for cp in row_copies(c):
            cp.wait()

    lane = lax.iota(jnp.int32, L)

    def compute_chunk(c):
        b = lax.bitwise_and(c, DEPTH - 1)
        u_t = urows_v.at[b]
        i_t = irows_v.at[b]
        accs = [jnp.zeros((L,), jnp.float32) for _ in range(4)]
        for d in range(D):
            dv = jnp.full((L,), d, jnp.int32)
            cu = plsc.load_gather(u_t, [lane, dv])
            cv = plsc.load_gather(i_t, [lane, dv])
            accs[d % 4] = accs[d % 4] + cu * cv
        dot = (accs[0] + accs[1]) + (accs[2] + accs[3])
        pref = dot * 0.2

        sl = pl.ds(c * L, L)
        bg = bg_v[sl]
        glv = gl_v[sl]
        post = bg + glv * 4.0
        t = post - 110.0
        health = jnp.exp(t * t * (-1.0 / 3200.0))

        a_sel = jnp.where(
            post >= 250.0, a0_v[sl],
            jnp.where(post > 180.0, a1_v[sl],
                      jnp.where(post >= 70.0, a2_v[sl],
                                jnp.where(post >= 55.0, a3_v[sl],
                                          a4_v[sl]))))
        logit = a_sel + a5_v[sl]
        alpha = 1.0 / (1.0 + jnp.exp(-logit))
        out_v[sl] = alpha * health + (1.0 - alpha) * pref

    for c in range(DEPTH):
        fire_chunk(jnp.int32(c))
    for c in scopies:
        c.wait()

    def steady_body(c, carry):
        wait_chunk(c)
        compute_chunk(c)
        fire_chunk(c + DEPTH)
        return carry

    lax.fori_loop(0, NCH - DEPTH, steady_body, 0)

    def drain_body(c, carry):
        wait_chunk(c)
        compute_chunk(c)
        return carry

    lax.fori_loop(NCH - DEPTH, NCH, drain_body, 0)

    pltpu.sync_copy(out_v, out_hbm.at[pl.ds(base, BPW)])


def kernel(user_indices, item_indices, blood_glucose, gl, user_emb, item_emb,
           a_hyper2, a_hyper1, a_normal, a_hypo1, a_hypo2, bias):
    ui = user_indices.astype(jnp.int32)
    ii = item_indices.astype(jnp.int32)
    return _sc_recommender(
        ui, ii, blood_glucose, gl, user_emb, item_emb,
        a_hyper2.T, a_hyper1.T, a_normal.T,
        a_hypo1.T, a_hypo2.T, bias.T)
